# baseline (speedup 1.0000x reference)
import functools

import jax
import jax.numpy as jnp
from jax import lax
from jax.experimental import pallas as pl
from jax.experimental.pallas import tpu as pltpu
from jax.experimental.pallas import tpu_sc as plsc

B = 4096
L = 200
V = 1000
D = 64

NC, NS, LANES = 2, 16, 16
NW = NC * NS
DT = 8
HU = L * DT * 2
HW = HU // NW
CHW = 16 * 8 * 128
NBUF = 2
VP = V + 8
ZROW = V


def _sc_body(table_hbm, tokT_hbm, ts_hbm, lut_hbm, out_hbm,
             table_v, lut_v, ts_v, tok_v, fid_v, stage_v, p0, p1):
    psems = (p0, p1)
    wid = lax.axis_index("s") * NC + lax.axis_index("c")
    hu0 = wid * HW

    pltpu.sync_copy(table_hbm, table_v)
    pltpu.sync_copy(lut_hbm, lut_v)
    pltpu.sync_copy(ts_hbm, ts_v)

    iota = lax.iota(jnp.int32, LANES)

    def step(och, carry):
        for half in range(NBUF):
            ch = och * NBUF + half
            hu = hu0 + ch
            l = hu >> 4
            dt8 = ((hu >> 1) & (DT - 1)) * 8
            boff = (hu & 1) * (16 * 128)
            svec = stage_v.at[half]

            @pl.when(((hu & 15) == 0) | (ch == 0))
            def _row():
                pltpu.sync_copy(tokT_hbm.at[pl.ds(l * B, B)], tok_v)

                @plsc.parallel_loop(0, B // LANES, unroll=4)
                def mkfid(g):
                    bo = g * LANES
                    tok = tok_v[pl.ds(bo, LANES)]
                    ids = plsc.load_gather(lut_v, [tok])
                    tsr = ts_v[pl.ds(bo, LANES)]
                    fid_v[pl.ds(bo, LANES)] = (
                        jnp.where(l < tsr, ids, ZROW) * D)

            @pl.when(och > 0)
            def _drain():
                pltpu.make_async_copy(
                    svec, out_hbm.at[pl.ds(0, CHW)], psems[half]).wait()

            @plsc.parallel_loop(0, 16 * 8, unroll=8)
            def inner(g):
                bo = boff + g * LANES
                fidd = fid_v[pl.ds(bo, LANES)] + dt8
                sbase = (g >> 3) * 1024 + (g & 7) * LANES + iota
                for ds_ in range(8):
                    cv = (iota + ds_) & 7
                    val = plsc.load_gather(table_v, [fidd + cv])
                    plsc.store_scatter(svec, [sbase + cv * 128], val)
            pltpu.async_copy(
                svec, out_hbm.at[pl.ds(hu * CHW, CHW)], psems[half])
        return carry

    lax.fori_loop(0, HW // NBUF, step, 0)
    for half in range(NBUF):
        pltpu.make_async_copy(
            stage_v.at[half], out_hbm.at[pl.ds(0, CHW)], psems[half]).wait()


_sc_gather = functools.partial(
    pl.kernel,
    out_type=jax.ShapeDtypeStruct((L * D * B,), jnp.float32),
    mesh=plsc.VectorSubcoreMesh(
        core_axis_name="c", subcore_axis_name="s",
        num_cores=NC, num_subcores=NS),
    scratch_types=[
        pltpu.VMEM((VP * D,), jnp.float32),
        pltpu.VMEM((V,), jnp.int32),
        pltpu.VMEM((B,), jnp.int32),
        pltpu.VMEM((B,), jnp.int32),
        pltpu.VMEM((B,), jnp.int32),
        pltpu.VMEM((NBUF, CHW), jnp.float32),
    ] + [pltpu.SemaphoreType.DMA] * NBUF,
    compiler_params=pltpu.CompilerParams(
        needs_layout_passes=False, use_tc_tiling_on_sc=False),
)(_sc_body)


def _tc_body(rl_ref, msl_ref, ts_ref, mask_ref):
    ts = jnp.minimum(jnp.minimum(rl_ref[...], msl_ref[...]), L).astype(jnp.int32)
    ts_ref[...] = ts
    col = lax.broadcasted_iota(jnp.int32, (B, L), 1)
    mask_ref[...] = col < ts


_tc_mask = pl.pallas_call(
    _tc_body,
    out_shape=(
        jax.ShapeDtypeStruct((B, 1), jnp.int32),
        jax.ShapeDtypeStruct((B, L), jnp.bool_),
    ),
)


def kernel(tokens, row_lengths, max_sequence_length, lookup_table, embeddings):
    msl = jnp.asarray(max_sequence_length, jnp.int32).reshape(1, 1)
    ts2d, mask = _tc_mask(row_lengths.reshape(B, 1).astype(jnp.int32), msl)
    time_steps = ts2d.reshape(B)
    table_ext = jnp.concatenate(
        [embeddings, jnp.zeros((VP - V, D), jnp.float32)], axis=0)
    tok_t = tokens.T.reshape(L * B)
    out_ph = _sc_gather(table_ext.reshape(VP * D), tok_t,
                        time_steps, lookup_table)
    out = out_ph.reshape(L, DT, NW, 8, 128).transpose(2, 4, 0, 1, 3)
    return out.reshape(B, L, D), mask, time_steps


# speedup vs baseline: 1.0309x; 1.0309x over previous
import functools

import jax
import jax.numpy as jnp
from jax import lax
from jax.experimental import pallas as pl
from jax.experimental.pallas import tpu as pltpu
from jax.experimental.pallas import tpu_sc as plsc

B = 4096
L = 200
V = 1000
D = 64

NC, NS, LANES = 2, 16, 16
NW = NC * NS
DT = 8
HU = L * DT * 2
HW = HU // NW
CHW = 16 * 8 * 128
NBUF = 2
VP = V + 8
ZROW = V


def _sc_body(table_hbm, tokT_hbm, ts_hbm, lut_hbm, out_hbm,
             table_v, lut_v, ts_v, tok_v, fid_v, stage_v, p0, p1, tsem):
    psems = (p0, p1)
    wid = lax.axis_index("s") * NC + lax.axis_index("c")
    hu0 = wid * HW

    pltpu.sync_copy(table_hbm, table_v)
    pltpu.sync_copy(lut_hbm, lut_v)
    pltpu.sync_copy(ts_hbm, ts_v)

    iota = lax.iota(jnp.int32, LANES)
    cvs = [(iota + ds_) & 7 for ds_ in range(8)]
    cv128s = [cv * 128 for cv in cvs]
    lmax = (hu0 + HW - 1) >> 4

    pltpu.sync_copy(tokT_hbm.at[pl.ds((hu0 >> 4) * B, B)],
                    tok_v.at[pl.ds(0, B)])

    def mkfid(l, toff):
        @plsc.parallel_loop(0, B // LANES, unroll=8)
        def _mk(g):
            bo = g * LANES
            tok = tok_v[pl.ds(toff + bo, LANES)]
            ids = plsc.load_gather(lut_v, [tok])
            tsr = ts_v[pl.ds(bo, LANES)]
            fid_v[pl.ds(bo, LANES)] = jnp.where(l < tsr, ids, ZROW) * D

    def step(och, carry):
        for half in range(NBUF):
            ch = och * NBUF + half
            hu = hu0 + ch
            l = hu >> 4
            dt8 = ((hu >> 1) & (DT - 1)) * 8
            boff = (hu & 1) * (16 * 128)
            svec = stage_v.at[half]
            tb = ((l ^ (hu0 >> 4)) & 1) * B

            @pl.when(((hu & 15) == 0) | (ch == 0))
            def _row():
                @pl.when(ch > 0)
                def _wtok():
                    pltpu.make_async_copy(
                        tokT_hbm.at[pl.ds(0, B)],
                        tok_v.at[pl.ds(0, B)], tsem).wait()
                mkfid(l, tb)
                @pl.when(l < lmax)
                def _pref():
                    pltpu.async_copy(
                        tokT_hbm.at[pl.ds((l + 1) * B, B)],
                        tok_v.at[pl.ds(B - tb, B)], tsem)

            @pl.when(och > 0)
            def _drain():
                pltpu.make_async_copy(
                    svec, out_hbm.at[pl.ds(0, CHW)], psems[half]).wait()

            tview = table_v.at[pl.ds(dt8, VP * D - 56)]

            @plsc.parallel_loop(0, 16 * 8, unroll=4)
            def inner(g):
                bo = boff + g * LANES
                fidd = fid_v[pl.ds(bo, LANES)]
                sbase = (g >> 3) * 1024 + (g & 7) * LANES + iota
                for ds_ in range(8):
                    val = plsc.load_gather(tview, [fidd + cvs[ds_]])
                    plsc.store_scatter(svec, [sbase + cv128s[ds_]], val)
            pltpu.async_copy(
                svec, out_hbm.at[pl.ds(hu * CHW, CHW)], psems[half])
        return carry

    lax.fori_loop(0, HW // NBUF, step, 0)
    for half in range(NBUF):
        pltpu.make_async_copy(
            stage_v.at[half], out_hbm.at[pl.ds(0, CHW)], psems[half]).wait()


_sc_gather = functools.partial(
    pl.kernel,
    out_type=jax.ShapeDtypeStruct((L * D * B,), jnp.float32),
    mesh=plsc.VectorSubcoreMesh(
        core_axis_name="c", subcore_axis_name="s",
        num_cores=NC, num_subcores=NS),
    scratch_types=[
        pltpu.VMEM((VP * D,), jnp.float32),
        pltpu.VMEM((V,), jnp.int32),
        pltpu.VMEM((B,), jnp.int32),
        pltpu.VMEM((2 * B,), jnp.int32),
        pltpu.VMEM((B,), jnp.int32),
        pltpu.VMEM((NBUF, CHW), jnp.float32),
    ] + [pltpu.SemaphoreType.DMA] * (NBUF + 1),
    compiler_params=pltpu.CompilerParams(
        needs_layout_passes=False, use_tc_tiling_on_sc=False),
)(_sc_body)


def _tc_body(rl_ref, msl_ref, ts_ref, mask_ref):
    ts = jnp.minimum(jnp.minimum(rl_ref[...], msl_ref[...]), L).astype(jnp.int32)
    ts_ref[...] = ts
    col = lax.broadcasted_iota(jnp.int32, (B, L), 1)
    mask_ref[...] = col < ts


_tc_mask = pl.pallas_call(
    _tc_body,
    out_shape=(
        jax.ShapeDtypeStruct((B, 1), jnp.int32),
        jax.ShapeDtypeStruct((B, L), jnp.bool_),
    ),
)


def kernel(tokens, row_lengths, max_sequence_length, lookup_table, embeddings):
    msl = jnp.asarray(max_sequence_length, jnp.int32).reshape(1, 1)
    ts2d, mask = _tc_mask(row_lengths.reshape(B, 1).astype(jnp.int32), msl)
    time_steps = ts2d.reshape(B)
    table_ext = jnp.concatenate(
        [embeddings, jnp.zeros((VP - V, D), jnp.float32)], axis=0)
    tok_t = tokens.T.reshape(L * B)
    out_ph = _sc_gather(table_ext.reshape(VP * D), tok_t,
                        time_steps, lookup_table)
    out = out_ph.reshape(L, DT, NW, 8, 128).transpose(2, 4, 0, 1, 3)
    return out.reshape(B, L, D), mask, time_steps


# speedup vs baseline: 1.2930x; 1.2543x over previous
import functools

import jax
import jax.numpy as jnp
from jax import lax
from jax.experimental import pallas as pl
from jax.experimental.pallas import tpu as pltpu
from jax.experimental.pallas import tpu_sc as plsc

B = 4096
L = 200
V = 1000
D = 64

NC, NS, LANES = 2, 16, 16
NW = NC * NS
DT = 8
UW = L * DT // NW
FUW = 32 * 8 * 128
QW = 8 * 8 * 128
CHN = UW * 2
NBUF = 2
VP = V + 8
ZROW = V


def _sc_body(table_hbm, tokT_hbm, ts_hbm, lut_hbm, out_hbm,
             table_v, lut_v, ts_v, tok_v, fid_v, stage_v, p0, p1, tsem):
    psems = (p0, p1)
    wid = lax.axis_index("s") * NC + lax.axis_index("c")

    pltpu.sync_copy(table_hbm, table_v)
    pltpu.sync_copy(lut_hbm, lut_v)
    pltpu.sync_copy(ts_hbm, ts_v)

    u0w = wid * UW
    iota = lax.iota(jnp.int32, LANES)
    cvs = [(iota + k) & 15 for k in range(16)]
    scos = [(cv >> 3) * QW + (cv & 7) * 128 for cv in cvs]
    lmax = (u0w + UW - 1) >> 3

    pltpu.sync_copy(tokT_hbm.at[pl.ds((u0w >> 3) * B, B)],
                    tok_v.at[pl.ds(0, B)])

    def mkfid(l, toff):
        @plsc.parallel_loop(0, B // LANES, unroll=8)
        def _mk(g):
            bo = g * LANES
            tok = tok_v[pl.ds(toff + bo, LANES)]
            ids = plsc.load_gather(lut_v, [tok])
            tsr = ts_v[pl.ds(bo, LANES)]
            fid_v[pl.ds(bo, LANES)] = jnp.where(l < tsr, ids, ZROW) * D

    def step(och, carry):
        for half in range(NBUF):
            ch = och * NBUF + half
            u0 = u0w + (ch >> 2) * 2
            q = ch & 3
            l = u0 >> 3
            dcol = (u0 & 7) * 8
            boff = q * (8 * 128)
            svec = stage_v.at[half]
            tb = ((l ^ (u0w >> 3)) & 1) * B

            @pl.when((((u0 & 7) == 0) & (q == 0)) | (ch == 0))
            def _row():
                @pl.when(ch > 0)
                def _wtok():
                    pltpu.make_async_copy(
                        tokT_hbm.at[pl.ds(0, B)],
                        tok_v.at[pl.ds(0, B)], tsem).wait()
                mkfid(l, tb)
                @pl.when(l < lmax)
                def _pref():
                    pltpu.async_copy(
                        tokT_hbm.at[pl.ds((l + 1) * B, B)],
                        tok_v.at[pl.ds(B - tb, B)], tsem)

            @pl.when(och > 0)
            def _drain():
                pltpu.make_async_copy(
                    svec.at[pl.ds(0, QW)],
                    out_hbm.at[pl.ds(0, QW)], psems[half]).wait()
                pltpu.make_async_copy(
                    svec.at[pl.ds(0, QW)],
                    out_hbm.at[pl.ds(0, QW)], psems[half]).wait()

            tview = table_v.at[pl.ds(dcol, VP * D - 48)]

            @plsc.parallel_loop(0, 8 * 8, unroll=4)
            def inner(g):
                bo = boff + g * LANES
                fidd = fid_v[pl.ds(bo, LANES)]
                sbase = (g >> 3) * 1024 + (g & 7) * LANES + iota
                for k in range(16):
                    val = plsc.load_gather(tview, [fidd + cvs[k]])
                    plsc.store_scatter(svec, [sbase + scos[k]], val)
            pltpu.async_copy(
                svec.at[pl.ds(0, QW)],
                out_hbm.at[pl.ds(u0 * FUW + q * QW, QW)], psems[half])
            pltpu.async_copy(
                svec.at[pl.ds(QW, QW)],
                out_hbm.at[pl.ds((u0 + 1) * FUW + q * QW, QW)], psems[half])
        return carry

    lax.fori_loop(0, CHN // NBUF, step, 0)
    for half in range(NBUF):
        for _ in range(2):
            pltpu.make_async_copy(
                stage_v.at[half].at[pl.ds(0, QW)],
                out_hbm.at[pl.ds(0, QW)], psems[half]).wait()


_sc_gather = functools.partial(
    pl.kernel,
    out_type=jax.ShapeDtypeStruct((L * D * B,), jnp.float32),
    mesh=plsc.VectorSubcoreMesh(
        core_axis_name="c", subcore_axis_name="s",
        num_cores=NC, num_subcores=NS),
    scratch_types=[
        pltpu.VMEM((VP * D,), jnp.float32),
        pltpu.VMEM((V,), jnp.int32),
        pltpu.VMEM((B,), jnp.int32),
        pltpu.VMEM((2 * B,), jnp.int32),
        pltpu.VMEM((B,), jnp.int32),
        pltpu.VMEM((NBUF, 2 * QW), jnp.float32),
    ] + [pltpu.SemaphoreType.DMA] * (NBUF + 1),
    compiler_params=pltpu.CompilerParams(
        needs_layout_passes=False, use_tc_tiling_on_sc=False),
)(_sc_body)


def _tc_body(rl_ref, msl_ref, ts_ref, mask_ref):
    ts = jnp.minimum(jnp.minimum(rl_ref[...], msl_ref[...]), L).astype(jnp.int32)
    ts_ref[...] = ts
    col = lax.broadcasted_iota(jnp.int32, (B, L), 1)
    mask_ref[...] = col < ts


_tc_mask = pl.pallas_call(
    _tc_body,
    out_shape=(
        jax.ShapeDtypeStruct((B, 1), jnp.int32),
        jax.ShapeDtypeStruct((B, L), jnp.bool_),
    ),
)


def kernel(tokens, row_lengths, max_sequence_length, lookup_table, embeddings):
    msl = jnp.asarray(max_sequence_length, jnp.int32).reshape(1, 1)
    ts2d, mask = _tc_mask(row_lengths.reshape(B, 1).astype(jnp.int32), msl)
    time_steps = ts2d.reshape(B)
    table_ext = jnp.concatenate(
        [embeddings, jnp.zeros((VP - V, D), jnp.float32)], axis=0)
    tok_t = tokens.T.reshape(L * B)
    out_ph = _sc_gather(table_ext.reshape(VP * D), tok_t,
                        time_steps, lookup_table)
    out = out_ph.reshape(L, DT, NW, 8, 128).transpose(2, 4, 0, 1, 3)
    return out.reshape(B, L, D), mask, time_steps


# speedup vs baseline: 1.3082x; 1.0117x over previous
import functools

import jax
import jax.numpy as jnp
from jax import lax
from jax.experimental import pallas as pl
from jax.experimental.pallas import tpu as pltpu
from jax.experimental.pallas import tpu_sc as plsc

B = 4096
L = 200
V = 1000
D = 64

NC, NS, LANES = 2, 16, 16
NW = NC * NS
DT = 8
UW = L * DT // NW
FUW = 32 * 8 * 128
QW = 8 * 8 * 128
CHN = UW * 2
NBUF = 2
VP = V + 8
ZROW = V


def _sc_body(table_hbm, tokT_hbm, ts_hbm, lut_hbm, msl_hbm, out_hbm,
             table_v, lut_v, ts_v, tok_v, fid_v, msl_v, stage_v, p0, p1, tsem):
    psems = (p0, p1)
    wid = lax.axis_index("s") * NC + lax.axis_index("c")

    pltpu.sync_copy(table_hbm, table_v)
    pltpu.sync_copy(lut_hbm, lut_v)
    pltpu.sync_copy(ts_hbm, ts_v)
    pltpu.sync_copy(msl_hbm, msl_v)

    u0w = wid * UW
    iota = lax.iota(jnp.int32, LANES)
    cvs = [(iota + k) & 15 for k in range(16)]
    scos = [(cv >> 3) * QW + (cv & 7) * 128 for cv in cvs]
    lmax = (u0w + UW - 1) >> 3

    pltpu.sync_copy(tokT_hbm.at[pl.ds((u0w >> 3) * B, B)],
                    tok_v.at[pl.ds(0, B)])

    mslv = msl_v[pl.ds(0, LANES)]

    def mkfid(l, toff):
        lok = l < mslv

        @plsc.parallel_loop(0, B // LANES, unroll=8)
        def _mk(g):
            bo = g * LANES
            tok = tok_v[pl.ds(toff + bo, LANES)]
            ids = plsc.load_gather(lut_v, [tok])
            rl = ts_v[pl.ds(bo, LANES)]
            fid_v[pl.ds(bo, LANES)] = (
                jnp.where((l < rl) & lok, ids, ZROW) * D)

    def step(och, carry):
        for half in range(NBUF):
            ch = och * NBUF + half
            u0 = u0w + (ch >> 2) * 2
            q = ch & 3
            l = u0 >> 3
            dcol = (u0 & 7) * 8
            boff = q * (8 * 128)
            svec = stage_v.at[half]
            tb = ((l ^ (u0w >> 3)) & 1) * B

            @pl.when((((u0 & 7) == 0) & (q == 0)) | (ch == 0))
            def _row():
                @pl.when(ch > 0)
                def _wtok():
                    pltpu.make_async_copy(
                        tokT_hbm.at[pl.ds(0, B)],
                        tok_v.at[pl.ds(0, B)], tsem).wait()
                mkfid(l, tb)
                @pl.when(l < lmax)
                def _pref():
                    pltpu.async_copy(
                        tokT_hbm.at[pl.ds((l + 1) * B, B)],
                        tok_v.at[pl.ds(B - tb, B)], tsem)

            @pl.when(och > 0)
            def _drain():
                pltpu.make_async_copy(
                    svec.at[pl.ds(0, QW)],
                    out_hbm.at[pl.ds(0, QW)], psems[half]).wait()
                pltpu.make_async_copy(
                    svec.at[pl.ds(0, QW)],
                    out_hbm.at[pl.ds(0, QW)], psems[half]).wait()

            tview = table_v.at[pl.ds(dcol, VP * D - 48)]

            @plsc.parallel_loop(0, 8 * 8, unroll=4)
            def inner(g):
                bo = boff + g * LANES
                fidd = fid_v[pl.ds(bo, LANES)]
                sbase = (g >> 3) * 1024 + (g & 7) * LANES + iota
                for k in range(16):
                    val = plsc.load_gather(tview, [fidd + cvs[k]])
                    plsc.store_scatter(svec, [sbase + scos[k]], val)
            pltpu.async_copy(
                svec.at[pl.ds(0, QW)],
                out_hbm.at[pl.ds(u0 * FUW + q * QW, QW)], psems[half])
            pltpu.async_copy(
                svec.at[pl.ds(QW, QW)],
                out_hbm.at[pl.ds((u0 + 1) * FUW + q * QW, QW)], psems[half])
        return carry

    lax.fori_loop(0, CHN // NBUF, step, 0)
    for half in range(NBUF):
        for _ in range(2):
            pltpu.make_async_copy(
                stage_v.at[half].at[pl.ds(0, QW)],
                out_hbm.at[pl.ds(0, QW)], psems[half]).wait()


_sc_gather = functools.partial(
    pl.kernel,
    out_type=jax.ShapeDtypeStruct((L * D * B,), jnp.float32),
    mesh=plsc.VectorSubcoreMesh(
        core_axis_name="c", subcore_axis_name="s",
        num_cores=NC, num_subcores=NS),
    scratch_types=[
        pltpu.VMEM((VP * D,), jnp.float32),
        pltpu.VMEM((V,), jnp.int32),
        pltpu.VMEM((B,), jnp.int32),
        pltpu.VMEM((2 * B,), jnp.int32),
        pltpu.VMEM((B,), jnp.int32),
        pltpu.VMEM((LANES,), jnp.int32),
        pltpu.VMEM((NBUF, 2 * QW), jnp.float32),
    ] + [pltpu.SemaphoreType.DMA] * (NBUF + 1),
    compiler_params=pltpu.CompilerParams(
        needs_layout_passes=False, use_tc_tiling_on_sc=False),
)(_sc_body)


def _tc_body(rl_ref, msl_ref, ts_ref, mask_ref):
    ts = jnp.minimum(jnp.minimum(rl_ref[...], msl_ref[...]), L).astype(jnp.int32)
    ts_ref[...] = ts
    col = lax.broadcasted_iota(jnp.int32, (B, L), 1)
    mask_ref[...] = col < ts


_tc_mask = pl.pallas_call(
    _tc_body,
    out_shape=(
        jax.ShapeDtypeStruct((B, 1), jnp.int32),
        jax.ShapeDtypeStruct((B, L), jnp.bool_),
    ),
)


def kernel(tokens, row_lengths, max_sequence_length, lookup_table, embeddings):
    msl = jnp.asarray(max_sequence_length, jnp.int32).reshape(1, 1)
    ts2d, mask = _tc_mask(row_lengths.reshape(B, 1).astype(jnp.int32), msl)
    time_steps = ts2d.reshape(B)
    table_ext = jnp.concatenate(
        [embeddings, jnp.zeros((VP - V, D), jnp.float32)], axis=0)
    tok_t = tokens.T.reshape(L * B)
    msl16 = jnp.broadcast_to(
        jnp.minimum(jnp.asarray(max_sequence_length, jnp.int32), L), (LANES,))
    out_ph = _sc_gather(table_ext.reshape(VP * D), tok_t,
                        row_lengths.astype(jnp.int32), lookup_table, msl16)
    out = out_ph.reshape(L, DT, NW, 8, 128).transpose(2, 4, 0, 1, 3)
    return out.reshape(B, L, D), mask, time_steps
